# Initial kernel scaffold; baseline (speedup 1.0000x reference)
import jax
import jax.numpy as jnp
from jax.experimental import pallas as pl


def kernel(coords, atom_types, radii, query_points, embed_table, rbf_centers):
    raise NotImplementedError("write your pallas kernel here")


# speedup vs baseline: 2.7885x; 2.7885x over previous
import functools

import jax
import jax.numpy as jnp
from jax import lax
from jax.experimental import pallas as pl
from jax.experimental.pallas import tpu as pltpu

_NUM_ATOM_TYPES = 100
_ATOM_EMBED_DIM = 16
_RBF_DIM = 16
_CUTOFF = 5.0
_MAX_NEIGHBORS = 32
_RBF_GAMMA = 1.0 / max(_CUTOFF / max(_RBF_DIM, 1), 1e-06) ** 2

_QT = 128
_LANE = 128
_INTERPRET = False


def _topk_body(q_ref, c_ref, dist_ref, idx_ref, mask_ref, sdist_ref):
    q = q_ref[0]
    c = c_ref[0]
    qc = lax.dot_general(q, c, (((0,), (0,)), ((), ())),
                         preferred_element_type=jnp.float32)
    q2 = jnp.sum(q * q, axis=0)[:, None]
    c2 = jnp.sum(c * c, axis=0)[None, :]
    d2 = q2 + c2 - 2.0 * qc
    d = jnp.sqrt(jnp.maximum(d2, 0.0))
    d = jnp.maximum(d, 1e-12)
    np_ = d.shape[1]
    iota = lax.broadcasted_iota(jnp.int32, (_QT, np_), 1)
    INF = jnp.float32(jnp.inf)
    NBIG = jnp.int32(2 ** 30)
    ms = []
    is_ = []
    dm = d
    for _ in range(_MAX_NEIGHBORS):
        m = jnp.min(dm, axis=1, keepdims=True)
        eq = dm == m
        im = jnp.min(jnp.where(eq, iota, NBIG), axis=1, keepdims=True)
        dm = jnp.where(iota == im, INF, dm)
        ms.append(m)
        is_.append(im)
    dists = jnp.concatenate(ms, axis=1)
    idx = jnp.concatenate(is_, axis=1)
    mask = dists <= _CUTOFF
    dist_ref[0] = dists
    idx_ref[0] = idx
    mask_ref[0] = mask
    sdist_ref[0] = jnp.where(mask, dists, 0.0)


def _run_topk(qT, cT):
    B, _, Q = qT.shape
    NP = cT.shape[2]
    K = _MAX_NEIGHBORS
    grid = (B, Q // _QT)
    out_shapes = (
        jax.ShapeDtypeStruct((B, Q, K), jnp.float32),
        jax.ShapeDtypeStruct((B, Q, K), jnp.int32),
        jax.ShapeDtypeStruct((B, Q, K), jnp.bool_),
        jax.ShapeDtypeStruct((B, Q, K), jnp.float32),
    )
    out_spec = pl.BlockSpec((1, _QT, K), lambda b, t: (b, t, 0))
    return pl.pallas_call(
        _topk_body,
        grid=grid,
        in_specs=[
            pl.BlockSpec((1, 3, _QT), lambda b, t: (b, 0, t)),
            pl.BlockSpec((1, 3, NP), lambda b, t: (b, 0, 0)),
        ],
        out_specs=(out_spec, out_spec, out_spec, out_spec),
        out_shape=out_shapes,
        interpret=_INTERPRET,
    )(qT, cT)


def kernel(coords, atom_types, radii, query_points, embed_table, rbf_centers):
    B, N, _ = coords.shape
    Q = query_points.shape[1]
    K = _MAX_NEIGHBORS
    NP = ((N + _LANE - 1) // _LANE) * _LANE

    cT = jnp.transpose(coords, (0, 2, 1))
    cT = jnp.pad(cT, ((0, 0), (0, 0), (0, NP - N)),
                 constant_values=1e9)
    qT = jnp.transpose(query_points, (0, 2, 1))

    sorted_dists, sorted_indices, neighbor_mask, safe_dists = _run_topk(qT, cT)

    def _bg(arr, idx):
        return jax.vmap(lambda a, i: a[i])(arr, idx)

    neighbor_coords = _bg(coords, sorted_indices)
    neighbor_radii = _bg(radii, sorted_indices)[..., None]
    zcol = jnp.zeros_like(neighbor_radii)
    neighbor_atom_types = _bg(atom_types, sorted_indices)
    neighbor_atom_emb = embed_table[neighbor_atom_types]
    rel_pos = query_points[:, :, None, :] - neighbor_coords
    rel_dist = sorted_dists[..., None]
    centers = rbf_centers.reshape(1, 1, 1, -1)
    rbf = jnp.exp(-_RBF_GAMMA * (rel_dist - centers) ** 2)
    features = jnp.concatenate([rel_pos, neighbor_radii, zcol, zcol, zcol,
                                neighbor_atom_emb, rbf, rel_dist], axis=-1)
    features = jnp.where(neighbor_mask[..., None], features, 0.0)
    return (features, neighbor_mask, sorted_indices, safe_dists)


# speedup vs baseline: 10.7004x; 3.8373x over previous
import functools

import jax
import jax.numpy as jnp
from jax import lax
from jax.experimental import pallas as pl
from jax.experimental.pallas import tpu as pltpu
from jax.experimental.pallas import tpu_sc as plsc

_NUM_ATOM_TYPES = 100
_ATOM_EMBED_DIM = 16
_RBF_DIM = 16
_CUTOFF = 5.0
_MAX_NEIGHBORS = 32
_RBF_GAMMA = 1.0 / max(_CUTOFF / max(_RBF_DIM, 1), 1e-06) ** 2

_QT = 128
_LANE = 128
_INTERPRET = False


def _topk_body(q_ref, c_ref, dist_ref, idx_ref, mask_ref, sdist_ref):
    q = q_ref[0]
    c = c_ref[0]
    qc = lax.dot_general(q, c, (((0,), (0,)), ((), ())),
                         preferred_element_type=jnp.float32)
    q2 = jnp.sum(q * q, axis=0)[:, None]
    c2 = jnp.sum(c * c, axis=0)[None, :]
    d2 = q2 + c2 - 2.0 * qc
    d = jnp.sqrt(jnp.maximum(d2, 0.0))
    d = jnp.maximum(d, 1e-12)
    np_ = d.shape[1]
    iota = lax.broadcasted_iota(jnp.int32, (_QT, np_), 1)
    INF = jnp.float32(jnp.inf)
    NBIG = jnp.int32(2 ** 30)
    ms = []
    is_ = []
    dm = d
    for _ in range(_MAX_NEIGHBORS):
        m = jnp.min(dm, axis=1, keepdims=True)
        eq = dm == m
        im = jnp.min(jnp.where(eq, iota, NBIG), axis=1, keepdims=True)
        dm = jnp.where(iota == im, INF, dm)
        ms.append(m)
        is_.append(im)
    dists = jnp.concatenate(ms, axis=1)
    idx = jnp.concatenate(is_, axis=1)
    mask = dists <= _CUTOFF
    dist_ref[0] = dists
    idx_ref[0] = idx
    mask_ref[0] = mask
    sdist_ref[0] = jnp.where(mask, dists, 0.0)


def _run_topk(qT, cT):
    B, _, Q = qT.shape
    NP = cT.shape[2]
    K = _MAX_NEIGHBORS
    grid = (B, Q // _QT)
    out_shapes = (
        jax.ShapeDtypeStruct((B, Q, K), jnp.float32),
        jax.ShapeDtypeStruct((B, Q, K), jnp.int32),
        jax.ShapeDtypeStruct((B, Q, K), jnp.bool_),
        jax.ShapeDtypeStruct((B, Q, K), jnp.float32),
    )
    out_spec = pl.BlockSpec((1, _QT, K), lambda b, t: (b, t, 0))
    return pl.pallas_call(
        _topk_body,
        grid=grid,
        in_specs=[
            pl.BlockSpec((1, 3, _QT), lambda b, t: (b, 0, t)),
            pl.BlockSpec((1, 3, NP), lambda b, t: (b, 0, 0)),
        ],
        out_specs=(out_spec, out_spec, out_spec, out_spec),
        out_shape=out_shapes,
        interpret=_INTERPRET,
    )(qT, cT)


_SC_NC = 2
_SC_NS = 16
_SC_L = 16
_NW = _SC_NC * _SC_NS
_FDIM = 40
_CHQ = 16


def _sc_features_body(cx_h, cy_h, cz_h, rad_h, typ_h, emb_h, ctr_h,
                      qx_h, qy_h, qz_h, idx_h, dst_h, feat_h,
                      cxv, cyv, czv, radv, typv, embv, ctrv,
                      qxv, qyv, qzv, idxv, dstv, fbuf, sem):
    K = _MAX_NEIGHBORS
    QW = qxv.shape[0]
    N = cxv.shape[0]
    wid = lax.axis_index("s") * _SC_NC + lax.axis_index("c")
    nq_total = _NW * QW
    b = (wid * QW) // (nq_total // 2)
    qbase = wid * QW

    pltpu.sync_copy(cx_h.at[b], cxv)
    pltpu.sync_copy(cy_h.at[b], cyv)
    pltpu.sync_copy(cz_h.at[b], czv)
    pltpu.sync_copy(rad_h.at[b], radv)
    pltpu.sync_copy(typ_h.at[b], typv)
    pltpu.sync_copy(emb_h, embv)
    pltpu.sync_copy(ctr_h, ctrv)
    pltpu.sync_copy(qx_h.at[pl.ds(qbase, QW)], qxv)
    pltpu.sync_copy(qy_h.at[pl.ds(qbase, QW)], qyv)
    pltpu.sync_copy(qz_h.at[pl.ds(qbase, QW)], qzv)
    pltpu.sync_copy(idx_h.at[pl.ds(qbase * K, QW * K)], idxv)
    pltpu.sync_copy(dst_h.at[pl.ds(qbase * K, QW * K)], dstv)

    lane = lax.broadcasted_iota(jnp.int32, (_SC_L,), 0)
    cutoff = jnp.float32(_CUTOFF)
    gamma = jnp.float32(_RBF_GAMMA)

    cbuf = [ctrv[pl.ds(e * _SC_L, _SC_L)] for e in range(_RBF_DIM)]

    for ch in range(QW // _CHQ):
        def per_query(qloc, carry):
            q = ch * _CHQ + qloc
            qsel = jnp.full((_SC_L,), q, jnp.int32)
            qx = plsc.load_gather(qxv, [qsel])
            qy = plsc.load_gather(qyv, [qsel])
            qz = plsc.load_gather(qzv, [qsel])
            for half in range(K // _SC_L):
                p0 = q * K + half * _SC_L
                idx16 = idxv[pl.ds(p0, _SC_L)]
                d16 = dstv[pl.ds(p0, _SC_L)]
                m16 = jnp.where(d16 <= cutoff, jnp.float32(1.0),
                                jnp.float32(0.0))
                cx16 = plsc.load_gather(cxv, [idx16])
                cy16 = plsc.load_gather(cyv, [idx16])
                cz16 = plsc.load_gather(czv, [idx16])
                rd16 = plsc.load_gather(radv, [idx16])
                tp16 = plsc.load_gather(typv, [idx16])
                floc = (qloc * K + half * _SC_L + lane) * _FDIM
                plsc.store_scatter(fbuf, [floc + 0], (qx - cx16) * m16)
                plsc.store_scatter(fbuf, [floc + 1], (qy - cy16) * m16)
                plsc.store_scatter(fbuf, [floc + 2], (qz - cz16) * m16)
                plsc.store_scatter(fbuf, [floc + 3], rd16 * m16)
                zero = jnp.zeros((_SC_L,), jnp.float32)
                plsc.store_scatter(fbuf, [floc + 4], zero)
                plsc.store_scatter(fbuf, [floc + 5], zero)
                plsc.store_scatter(fbuf, [floc + 6], zero)
                tbase = tp16 * _ATOM_EMBED_DIM
                for e in range(_ATOM_EMBED_DIM):
                    ev = plsc.load_gather(embv, [tbase + e])
                    plsc.store_scatter(fbuf, [floc + (7 + e)], ev * m16)
                for e in range(_RBF_DIM):
                    delta = d16 - cbuf[e]
                    rv = jnp.exp(-gamma * delta * delta)
                    plsc.store_scatter(fbuf, [floc + (23 + e)], rv * m16)
                plsc.store_scatter(fbuf, [floc + 39], d16 * m16)
            return carry
        lax.fori_loop(0, _CHQ, per_query, 0)
        wbase = (qbase + ch * _CHQ) * K * _FDIM
        pltpu.sync_copy(fbuf, feat_h.at[pl.ds(wbase, _CHQ * K * _FDIM)])


def _run_sc_features(coords, atom_types, radii, query_points, embed_table,
                     rbf_centers, sorted_indices, sorted_dists):
    B, N, _ = coords.shape
    Q = query_points.shape[1]
    K = _MAX_NEIGHBORS
    QW = (B * Q) // _NW
    cT = jnp.transpose(coords, (0, 2, 1))
    cx, cy, cz = cT[:, 0, :], cT[:, 1, :], cT[:, 2, :]
    qT = jnp.transpose(query_points, (2, 0, 1)).reshape(3, B * Q)
    typ = atom_types.astype(jnp.int32)
    emb = embed_table.reshape(-1)
    idxf = sorted_indices.reshape(-1)
    dstf = sorted_dists.reshape(-1)

    mesh = plsc.VectorSubcoreMesh(core_axis_name="c", subcore_axis_name="s")
    ctr_rep = jnp.repeat(rbf_centers, _SC_L)
    fn = functools.partial(
        pl.kernel,
        mesh=mesh,
        compiler_params=pltpu.CompilerParams(needs_layout_passes=False),
        out_type=jax.ShapeDtypeStruct((B * Q * K * _FDIM,), jnp.float32),
        scratch_types=[
            pltpu.VMEM((N,), jnp.float32),
            pltpu.VMEM((N,), jnp.float32),
            pltpu.VMEM((N,), jnp.float32),
            pltpu.VMEM((N,), jnp.float32),
            pltpu.VMEM((N,), jnp.int32),
            pltpu.VMEM((_NUM_ATOM_TYPES * _ATOM_EMBED_DIM,), jnp.float32),
            pltpu.VMEM((_RBF_DIM * _SC_L,), jnp.float32),
            pltpu.VMEM((QW,), jnp.float32),
            pltpu.VMEM((QW,), jnp.float32),
            pltpu.VMEM((QW,), jnp.float32),
            pltpu.VMEM((QW * K,), jnp.int32),
            pltpu.VMEM((QW * K,), jnp.float32),
            pltpu.VMEM((_CHQ * K * _FDIM,), jnp.float32),
            pltpu.SemaphoreType.DMA,
        ],
    )(_sc_features_body)
    feat = fn(cx, cy, cz, radii, typ, emb, ctr_rep,
              qT[0], qT[1], qT[2], idxf, dstf)
    return feat.reshape(B, Q, K, _FDIM)


def kernel(coords, atom_types, radii, query_points, embed_table, rbf_centers):
    B, N, _ = coords.shape
    Q = query_points.shape[1]
    NP = ((N + _LANE - 1) // _LANE) * _LANE

    cT = jnp.transpose(coords, (0, 2, 1))
    cT = jnp.pad(cT, ((0, 0), (0, 0), (0, NP - N)),
                 constant_values=1e9)
    qT = jnp.transpose(query_points, (0, 2, 1))

    sorted_dists, sorted_indices, neighbor_mask, safe_dists = _run_topk(qT, cT)
    features = _run_sc_features(coords, atom_types, radii, query_points,
                                embed_table, rbf_centers,
                                sorted_indices, sorted_dists)
    return (features, neighbor_mask, sorted_indices, safe_dists)
